# Initial kernel scaffold; baseline (speedup 1.0000x reference)
import jax
import jax.numpy as jnp
from jax.experimental import pallas as pl


def kernel(features, labels):
    raise NotImplementedError("write your pallas kernel here")


# speedup vs baseline: 4.0332x; 4.0332x over previous
import functools

import jax
import jax.numpy as jnp
from jax import lax
from jax.experimental import pallas as pl
from jax.experimental.pallas import tpu as pltpu
from jax.experimental.pallas import tpu_sc as plsc

_C = 100
_CP = 128
_A = 128
_L = 16
_NC = 2
_NS = 16
_NW = _NC * _NS


def _sc_partials(features, labels):
  n = features.shape[0]
  rows_per_w = n // _NW
  chunk = 400
  nch = rows_per_w // chunk
  vregs_per_row = _A // _L

  mesh = plsc.VectorSubcoreMesh(core_axis_name="c", subcore_axis_name="s")

  @functools.partial(
      pl.kernel,
      out_type=[
          jax.ShapeDtypeStruct((_NW, _CP * _A), jnp.float32),
          jax.ShapeDtypeStruct((_NW, _CP * _L), jnp.float32),
      ],
      mesh=mesh,
      compiler_params=pltpu.CompilerParams(needs_layout_passes=False),
      scratch_types=[
          pltpu.VMEM((rows_per_w + _L,), jnp.int32),
          pltpu.VMEM((chunk, _A), jnp.float32),
          pltpu.VMEM((_CP * _A,), jnp.float32),
          pltpu.VMEM((_CP * _L,), jnp.float32),
      ],
  )
  def k(feat_hbm, lab_hbm, out_sums, out_cnt, lab_v, buf, acc, cnt):
    wid = lax.axis_index("s") * _NC + lax.axis_index("c")
    base = wid * rows_per_w

    zeros = jnp.zeros((_L,), jnp.float32)

    def zero_acc(i, _):
      acc[pl.ds(i * _L, _L)] = zeros
      return 0
    lax.fori_loop(0, (_CP * _A) // _L, zero_acc, 0)

    def zero_cnt(i, _):
      cnt[pl.ds(i * _L, _L)] = zeros
      return 0
    lax.fori_loop(0, _CP, zero_cnt, 0)

    pltpu.sync_copy(lab_hbm.at[pl.ds(base, rows_per_w)],
                    lab_v.at[pl.ds(0, rows_per_w)])

    lane = lax.iota(jnp.int32, _L)
    ones = jnp.ones((_L,), jnp.float32)

    def cnt_body(g, _):
      lab16 = lab_v[pl.ds(g * _L, _L)]
      plsc.addupdate_scatter(cnt, [lab16 * _L + lane], ones)
      return 0
    lax.fori_loop(0, rows_per_w // _L, cnt_body, 0)

    def chunk_body(g, _):
      pltpu.sync_copy(feat_hbm.at[pl.ds(base + g * chunk, chunk)], buf)

      def row_body(r, _):
        lab = lab_v[pl.ds(g * chunk + r, _L)][0]
        off = lab * _A
        for j in range(vregs_per_row):
          v = buf[r, pl.ds(j * _L, _L)]
          plsc.addupdate(acc.at[pl.ds(off + j * _L, _L)], v)
        return 0
      lax.fori_loop(0, chunk, row_body, 0)
      return 0
    lax.fori_loop(0, nch, chunk_body, 0)

    pltpu.sync_copy(acc, out_sums.at[wid])
    pltpu.sync_copy(cnt, out_cnt.at[wid])

  return k(features, labels)


def _combine_kernel(sums_ref, cnt_ref, out_ref):
  s = jnp.sum(sums_ref[...], axis=0)
  c = jnp.sum(cnt_ref[...], axis=(0, 2))
  denom = jnp.where(c == 0.0, 1.0, c)
  out_ref[...] = s / denom[:, None]


def _combine(partial_sums, partial_cnt):
  return pl.pallas_call(
      _combine_kernel,
      out_shape=jax.ShapeDtypeStruct((_CP, _A), jnp.float32),
  )(partial_sums, partial_cnt)


@jax.jit
def kernel(features, labels):
  partial_sums, partial_cnt = _sc_partials(features, labels)
  partial_sums = partial_sums.reshape(_NW, _CP, _A)
  partial_cnt = partial_cnt.reshape(_NW, _CP, _L)
  avg = _combine(partial_sums, partial_cnt)
  return lax.stop_gradient(avg[:_C])


# speedup vs baseline: 14.1033x; 3.4968x over previous
import functools

import jax
import jax.numpy as jnp
from jax import lax
from jax.experimental import pallas as pl
from jax.experimental.pallas import tpu as pltpu
from jax.experimental.pallas import tpu_sc as plsc

_C = 100
_CP = 128
_A = 128
_L = 16
_NC = 2
_NS = 16
_NW = _NC * _NS


def _sc_partials(features, labels):
  n = features.shape[0]
  rows_per_w = n // _NW
  chunk = 200
  nch = rows_per_w // chunk
  vregs_per_row = _A // _L

  mesh = plsc.VectorSubcoreMesh(core_axis_name="c", subcore_axis_name="s")

  @functools.partial(
      pl.kernel,
      out_type=[
          jax.ShapeDtypeStruct((_NW, _CP * _A), jnp.float32),
          jax.ShapeDtypeStruct((_NW, _CP * _L), jnp.float32),
      ],
      mesh=mesh,
      compiler_params=pltpu.CompilerParams(needs_layout_passes=False),
      scratch_types=[
          pltpu.VMEM((rows_per_w + _L,), jnp.int32),
          pltpu.VMEM_SHARED((_NS * rows_per_w,), jnp.int32),
          pltpu.SMEM((chunk,), jnp.int32),
          pltpu.VMEM((chunk, _A), jnp.float32),
          pltpu.VMEM((chunk, _A), jnp.float32),
          pltpu.VMEM((_CP * _A,), jnp.float32),
          pltpu.VMEM((_CP * _L,), jnp.float32),
          pltpu.SemaphoreType.DMA,
          pltpu.SemaphoreType.DMA,
      ],
  )
  def k(feat_hbm, lab_hbm, out_sums, out_cnt,
        lab_v, lab_sh, lab_s, buf0, buf1, acc, cnt, sem0, sem1):
    cid = lax.axis_index("c")
    sid = lax.axis_index("s")
    wid = cid * _NS + sid
    base = wid * rows_per_w

    @pl.when(sid == 0)
    def _():
      pltpu.sync_copy(lab_hbm.at[pl.ds(cid * _NS * rows_per_w,
                                       _NS * rows_per_w)], lab_sh)
    plsc.subcore_barrier()

    zeros = jnp.zeros((_L,), jnp.float32)

    def zero_acc(i, _):
      acc[pl.ds(i * _L, _L)] = zeros
      return 0
    lax.fori_loop(0, (_CP * _A) // _L, zero_acc, 0)

    def zero_cnt(i, _):
      cnt[pl.ds(i * _L, _L)] = zeros
      return 0
    lax.fori_loop(0, _CP, zero_cnt, 0)

    pltpu.sync_copy(lab_hbm.at[pl.ds(base, rows_per_w)],
                    lab_v.at[pl.ds(0, rows_per_w)])

    lane = lax.iota(jnp.int32, _L)
    ones = jnp.ones((_L,), jnp.float32)

    def cnt_body(g, _):
      lab16 = lab_v[pl.ds(g * _L, _L)]
      plsc.addupdate_scatter(cnt, [lab16 * _L + lane], ones)
      return 0
    lax.fori_loop(0, rows_per_w // _L, cnt_body, 0)

    def start_dma(g, buf, sem):
      return pltpu.async_copy(
          feat_hbm.at[pl.ds(base + g * chunk, chunk)], buf, sem)

    def wait_dma(buf, sem):
      pltpu.make_async_copy(feat_hbm.at[pl.ds(base, chunk)], buf, sem).wait()

    def process(g, buf):
      pltpu.sync_copy(lab_sh.at[pl.ds(sid * rows_per_w + g * chunk, chunk)],
                      lab_s)

      def row_body(r, _):
        lab = lab_s[r]
        off = lab * _A
        vs = [buf[r, pl.ds(j * _L, _L)] for j in range(vregs_per_row)]
        for j in range(vregs_per_row):
          plsc.addupdate(acc.at[pl.ds(off + j * _L, _L)], vs[j])
        return 0
      lax.fori_loop(0, chunk, row_body, 0, unroll=2)

    start_dma(0, buf0, sem0)

    def chunk_body(h, _):
      g = h * 2
      start_dma(g + 1, buf1, sem1)
      wait_dma(buf0, sem0)
      process(g, buf0)

      @pl.when(g + 2 < nch)
      def _():
        start_dma(g + 2, buf0, sem0)
      wait_dma(buf1, sem1)
      process(g + 1, buf1)
      return 0
    lax.fori_loop(0, nch // 2, chunk_body, 0)

    pltpu.sync_copy(acc, out_sums.at[wid])
    pltpu.sync_copy(cnt, out_cnt.at[wid])

  return k(features, labels)


def _combine_kernel(sums_ref, cnt_ref, out_ref):
  s = jnp.sum(sums_ref[...], axis=0)
  c = jnp.sum(cnt_ref[...], axis=(0, 2))
  denom = jnp.where(c == 0.0, 1.0, c)
  out_ref[...] = s / denom[:, None]


def _combine(partial_sums, partial_cnt):
  return pl.pallas_call(
      _combine_kernel,
      out_shape=jax.ShapeDtypeStruct((_CP, _A), jnp.float32),
  )(partial_sums, partial_cnt)


@jax.jit
def kernel(features, labels):
  partial_sums, partial_cnt = _sc_partials(features, labels)
  partial_sums = partial_sums.reshape(_NW, _CP, _A)
  partial_cnt = partial_cnt.reshape(_NW, _CP, _L)
  avg = _combine(partial_sums, partial_cnt)
  return lax.stop_gradient(avg[:_C])


# speedup vs baseline: 16.3912x; 1.1622x over previous
import functools

import jax
import jax.numpy as jnp
from jax import lax
from jax.experimental import pallas as pl
from jax.experimental.pallas import tpu as pltpu
from jax.experimental.pallas import tpu_sc as plsc

_C = 100
_CP = 128
_A = 128
_L = 16
_NC = 2
_NS = 16
_NW = _NC * _NS
_B = 100


def _sc_partials(features, labels, labels3d):
  n = features.shape[0]
  rows_per_w = n // _NW
  chunk = 200
  nch = rows_per_w // chunk
  nbatch = rows_per_w // _B

  mesh = plsc.VectorSubcoreMesh(core_axis_name="c", subcore_axis_name="s")

  @functools.partial(
      pl.kernel,
      out_type=[
          jax.ShapeDtypeStruct((_NC, _CP, _A), jnp.float32),
          jax.ShapeDtypeStruct((_NW, _CP * _L), jnp.float32),
      ],
      mesh=mesh,
      compiler_params=pltpu.CompilerParams(needs_layout_passes=False),
      scratch_types=[
          pltpu.VMEM((rows_per_w + _L,), jnp.int32),
          pltpu.VMEM((nbatch, _B), jnp.int32),
          pltpu.VMEM((chunk, _A), jnp.float32),
          pltpu.VMEM((chunk, _A), jnp.float32),
          pltpu.VMEM((_CP, _A), jnp.float32),
          pltpu.VMEM_SHARED((_CP, _A), jnp.float32),
          pltpu.VMEM((_CP * _L,), jnp.float32),
          pltpu.SemaphoreType.DMA,
          pltpu.SemaphoreType.DMA,
      ],
  )
  def k(feat_hbm, lab_hbm, lab3d_hbm, out_sums, out_cnt,
        lab_v, lab2d, buf0, buf1, zbuf, acc_sh, cnt, sem0, sem1):
    cid = lax.axis_index("c")
    sid = lax.axis_index("s")
    wid = cid * _NS + sid
    base = wid * rows_per_w

    zeros = jnp.zeros((_L,), jnp.float32)

    @pl.when(sid == 0)
    def _():
      def zero_acc(i, _):
        for j in range(_A // _L):
          zbuf[i, pl.ds(j * _L, _L)] = zeros
        return 0
      lax.fori_loop(0, _CP, zero_acc, 0)
      pltpu.sync_copy(zbuf, acc_sh)
    plsc.subcore_barrier()

    def zero_cnt(i, _):
      cnt[pl.ds(i * _L, _L)] = zeros
      return 0
    lax.fori_loop(0, _CP, zero_cnt, 0)

    pltpu.sync_copy(lab_hbm.at[pl.ds(base, rows_per_w)],
                    lab_v.at[pl.ds(0, rows_per_w)])
    pltpu.sync_copy(lab3d_hbm.at[wid], lab2d)

    lane = lax.iota(jnp.int32, _L)
    ones = jnp.ones((_L,), jnp.float32)

    def cnt_body(g, _):
      lab16 = lab_v[pl.ds(g * _L, _L)]
      plsc.addupdate_scatter(cnt, [lab16 * _L + lane], ones)
      return 0
    lax.fori_loop(0, rows_per_w // _L, cnt_body, 0)

    def start_dma(g, buf, sem):
      return pltpu.async_copy(
          feat_hbm.at[pl.ds(base + g * chunk, chunk)], buf, sem)

    def wait_dma(buf, sem):
      pltpu.make_async_copy(feat_hbm.at[pl.ds(base, chunk)], buf, sem).wait()

    def process(g, buf):
      for b in range(chunk // _B):
        pltpu.sync_copy(buf.at[pl.ds(b * _B, _B)],
                        acc_sh.at[lab2d.at[g * (chunk // _B) + b]],
                        add=True)

    start_dma(0, buf0, sem0)
    start_dma(1, buf1, sem1)

    def chunk_body(h, _):
      g = h * 2
      wait_dma(buf0, sem0)
      process(g, buf0)

      @pl.when(g + 2 < nch)
      def _():
        start_dma(g + 2, buf0, sem0)
      wait_dma(buf1, sem1)
      process(g + 1, buf1)

      @pl.when(g + 3 < nch)
      def _():
        start_dma(g + 3, buf1, sem1)
      return 0
    lax.fori_loop(0, nch // 2, chunk_body, 0)

    plsc.subcore_barrier()
    @pl.when(sid == 0)
    def _():
      pltpu.sync_copy(acc_sh, out_sums.at[cid])
    pltpu.sync_copy(cnt, out_cnt.at[wid])

  return k(features, labels, labels3d)


def _combine_kernel(sums_ref, cnt_ref, out_ref):
  s = sums_ref[0] + sums_ref[1]
  c = jnp.sum(cnt_ref[...], axis=(0, 2))
  denom = jnp.where(c == 0.0, 1.0, c)
  out_ref[...] = s / denom[:, None]


def _combine(partial_sums, partial_cnt):
  return pl.pallas_call(
      _combine_kernel,
      out_shape=jax.ShapeDtypeStruct((_CP, _A), jnp.float32),
  )(partial_sums, partial_cnt)


@jax.jit
def kernel(features, labels):
  labels3d = labels.reshape(_NW, -1, _B)
  partial_sums, partial_cnt = _sc_partials(features, labels, labels3d)
  partial_cnt = partial_cnt.reshape(_NW, _CP, _L)
  avg = _combine(partial_sums, partial_cnt)
  return lax.stop_gradient(avg[:_C])


# speedup vs baseline: 18.0174x; 1.0992x over previous
import functools

import jax
import jax.numpy as jnp
from jax import lax
from jax.experimental import pallas as pl
from jax.experimental.pallas import tpu as pltpu
from jax.experimental.pallas import tpu_sc as plsc

_C = 100
_CP = 128
_A = 128
_L = 16
_NC = 2
_NS = 16
_NW = _NC * _NS
_B = 80
_NBUF = 5


def _sc_partials(features, labels, labels3d):
  n = features.shape[0]
  rows_per_w = n // _NW
  chunk = _B
  nch = rows_per_w // chunk
  assert nch % _NBUF == 0

  mesh = plsc.VectorSubcoreMesh(core_axis_name="c", subcore_axis_name="s")

  @functools.partial(
      pl.kernel,
      out_type=[
          jax.ShapeDtypeStruct((_NC, _CP, _A), jnp.float32),
          jax.ShapeDtypeStruct((_NW, _CP * _L), jnp.float32),
      ],
      mesh=mesh,
      compiler_params=pltpu.CompilerParams(needs_layout_passes=False),
      scratch_types=[
          pltpu.VMEM((rows_per_w + _L,), jnp.int32),
          pltpu.VMEM((nch, _B), jnp.int32),
          [pltpu.VMEM((chunk, _A), jnp.float32) for _ in range(_NBUF)],
          pltpu.VMEM((_CP, _A), jnp.float32),
          pltpu.VMEM_SHARED((_CP, _A), jnp.float32),
          pltpu.VMEM((_CP * _L,), jnp.float32),
          [pltpu.SemaphoreType.DMA for _ in range(_NBUF)],
      ],
  )
  def k(feat_hbm, lab_hbm, lab3d_hbm, out_sums, out_cnt,
        lab_v, lab2d, bufs, zbuf, acc_sh, cnt, sems):
    cid = lax.axis_index("c")
    sid = lax.axis_index("s")
    wid = cid * _NS + sid
    base = wid * rows_per_w

    zeros = jnp.zeros((_L,), jnp.float32)

    @pl.when(sid == 0)
    def _():
      def zero_acc(i, _):
        for j in range(_A // _L):
          zbuf[i, pl.ds(j * _L, _L)] = zeros
        return 0
      lax.fori_loop(0, _CP, zero_acc, 0)
      pltpu.sync_copy(zbuf, acc_sh)
    plsc.subcore_barrier()

    def zero_cnt(i, _):
      cnt[pl.ds(i * _L, _L)] = zeros
      return 0
    lax.fori_loop(0, _CP, zero_cnt, 0)

    pltpu.sync_copy(lab_hbm.at[pl.ds(base, rows_per_w)],
                    lab_v.at[pl.ds(0, rows_per_w)])
    pltpu.sync_copy(lab3d_hbm.at[wid], lab2d)

    def start_dma(g, buf, sem):
      return pltpu.async_copy(
          feat_hbm.at[pl.ds(base + g * chunk, chunk)], buf, sem)

    def wait_dma(buf, sem):
      pltpu.make_async_copy(feat_hbm.at[pl.ds(base, chunk)], buf, sem).wait()

    for b in range(_NBUF):
      start_dma(b, bufs[b], sems[b])

    lane = lax.iota(jnp.int32, _L)
    ones = jnp.ones((_L,), jnp.float32)

    def cnt_body(g, _):
      lab16 = lab_v[pl.ds(g * _L, _L)]
      plsc.addupdate_scatter(cnt, [lab16 * _L + lane], ones)
      return 0
    lax.fori_loop(0, rows_per_w // _L, cnt_body, 0)

    def chunk_body(h, _):
      for b in range(_NBUF):
        g = h * _NBUF + b
        wait_dma(bufs[b], sems[b])
        pltpu.sync_copy(bufs[b], acc_sh.at[lab2d.at[g]], add=True)

        @pl.when(g + _NBUF < nch)
        def _():
          start_dma(g + _NBUF, bufs[b], sems[b])
      return 0
    lax.fori_loop(0, nch // _NBUF, chunk_body, 0)

    plsc.subcore_barrier()
    @pl.when(sid == 0)
    def _():
      pltpu.sync_copy(acc_sh, out_sums.at[cid])
    pltpu.sync_copy(cnt, out_cnt.at[wid])

  return k(features, labels, labels3d)


def _combine_kernel(sums_ref, cnt_ref, out_ref):
  s = sums_ref[0] + sums_ref[1]
  c = jnp.sum(cnt_ref[...], axis=(0, 2))
  denom = jnp.where(c == 0.0, 1.0, c)
  out_ref[...] = s / denom[:, None]


def _combine(partial_sums, partial_cnt):
  return pl.pallas_call(
      _combine_kernel,
      out_shape=jax.ShapeDtypeStruct((_CP, _A), jnp.float32),
  )(partial_sums, partial_cnt)


@jax.jit
def kernel(features, labels):
  labels3d = labels.reshape(_NW, -1, _B)
  partial_sums, partial_cnt = _sc_partials(features, labels, labels3d)
  partial_cnt = partial_cnt.reshape(_NW, _CP, _L)
  avg = _combine(partial_sums, partial_cnt)
  return lax.stop_gradient(avg[:_C])
